# Initial kernel scaffold; baseline (speedup 1.0000x reference)
import jax
import jax.numpy as jnp
from jax.experimental import pallas as pl


def kernel(x, angles):
    raise NotImplementedError("write your pallas kernel here")


# speedup vs baseline: 6.0651x; 6.0651x over previous
import functools

import numpy as np
import jax
import jax.numpy as jnp
from jax import lax
from jax.experimental import pallas as pl
from jax.experimental.pallas import tpu as pltpu
from jax.experimental.pallas import tpu_sc as plsc

DIM_F = 1024
N_STAGES = 10
LANES = 16
NC, NS = 2, 16
NW = NC * NS
ROWS = 4 * 4096
ROWS_PER_W = ROWS // NW
CH = 8
N_CHUNKS = ROWS_PER_W // CH
N_PAIR_VREGS = DIM_F // (2 * LANES)
N_VREGS = DIM_F // LANES


def _build_tables():
    aidx = np.zeros((N_STAGES, DIM_F), np.int32)
    sgn = np.zeros((N_STAGES, DIM_F), np.float32)
    p = np.arange(DIM_F)
    for s in range(N_STAGES):
        stride = 1 << s
        aidx[s] = (((p >> (s + 1)) << s) | (p & (stride - 1))).astype(np.int32)
        sgn[s] = np.where((p & stride) != 0, 1.0, -1.0).astype(np.float32)
    return aidx, sgn


_AIDX, _SGN = _build_tables()


def _coef_body(a_ref, g_ref, c_ref, s_ref):
    a = a_ref[...]
    c_ref[...] = jnp.cos(a)
    s_ref[...] = jnp.sin(a) * g_ref[...]


def _make_coefs(angles):
    a_feat = jnp.take_along_axis(angles, jnp.asarray(_AIDX), axis=1)
    g_feat = jnp.asarray(_SGN)
    pad = ((0, 16 - N_STAGES), (0, 0))
    a16 = jnp.pad(a_feat, pad)
    g16 = jnp.pad(g_feat, pad)
    c16, s16 = pl.pallas_call(
        _coef_body,
        out_shape=(
            jax.ShapeDtypeStruct((16, DIM_F), jnp.float32),
            jax.ShapeDtypeStruct((16, DIM_F), jnp.float32),
        ),
    )(a16, g16)
    return c16[:N_STAGES], s16[:N_STAGES]


_MESH = plsc.VectorSubcoreMesh(
    core_axis_name="c", subcore_axis_name="s", num_cores=NC, num_subcores=NS
)


@functools.partial(
    pl.kernel,
    out_type=jax.ShapeDtypeStruct((ROWS, DIM_F), jnp.float32),
    mesh=_MESH,
    scratch_types=[
        pltpu.VMEM((CH, DIM_F), jnp.float32),
        pltpu.VMEM((N_STAGES, DIM_F), jnp.float32),
        pltpu.VMEM((N_STAGES, DIM_F), jnp.float32),
    ],
    compiler_params=pltpu.CompilerParams(needs_layout_passes=False),
)
def _butterfly_sc(x_hbm, c_hbm, s_hbm, out_hbm, buf, cc, ss):
    wid = lax.axis_index("s") * NC + lax.axis_index("c")
    base = wid * ROWS_PER_W
    pltpu.sync_copy(c_hbm, cc)
    pltpu.sync_copy(s_hbm, ss)

    def chunk_body(g, carry):
        row0 = base + g * CH
        pltpu.sync_copy(x_hbm.at[pl.ds(row0, CH)], buf)

        for st in range(N_STAGES):
            stride = 1 << st
            if stride >= LANES:
                svr = stride // LANES
                b = st - 4

                def pair_body(k, c2, svr=svr, b=b, st=st):
                    va = ((k >> b) << (b + 1)) | (k & (svr - 1))
                    oa = va * LANES
                    ob = oa + svr * LANES
                    cv = cc[st, pl.ds(ob, LANES)]
                    sn = ss[st, pl.ds(ob, LANES)]
                    for r in range(CH):
                        av = buf[r, pl.ds(oa, LANES)]
                        bv = buf[r, pl.ds(ob, LANES)]
                        buf[r, pl.ds(oa, LANES)] = cv * av - sn * bv
                        buf[r, pl.ds(ob, LANES)] = cv * bv + sn * av
                    return c2

                lax.fori_loop(0, N_PAIR_VREGS, pair_body, 0)
            else:
                perm = lax.iota(jnp.int32, LANES) ^ stride

                zeros = lax.iota(jnp.int32, LANES) * 0

                def vreg_body(v, c2, perm=perm, zeros=zeros, st=st):
                    off = v * LANES
                    cv = cc[st, pl.ds(off, LANES)]
                    sn = ss[st, pl.ds(off, LANES)]
                    pidx = perm + off
                    for r in range(CH):
                        xv = buf[r, pl.ds(off, LANES)]
                        xp = plsc.load_gather(buf, [zeros + r, pidx])
                        buf[r, pl.ds(off, LANES)] = cv * xv + sn * xp
                    return c2

                lax.fori_loop(0, N_VREGS, vreg_body, 0)

        pltpu.sync_copy(buf, out_hbm.at[pl.ds(row0, CH)])
        return carry

    lax.fori_loop(0, N_CHUNKS, chunk_body, 0)


def kernel(x, angles):
    orig_shape = x.shape
    x2 = x.reshape(ROWS, DIM_F)
    cc, ss = _make_coefs(angles)
    out = _butterfly_sc(x2, cc, ss)
    return out.reshape(orig_shape)
